# baseline (speedup 1.0000x reference)
import jax
import jax.numpy as jnp
from jax import lax
from jax.experimental import pallas as pl
from jax.experimental.pallas import tpu as pltpu
from jax.experimental.pallas import tpu_sc as plsc

VOCAB_ = 100001
D_ = 300
B_ = 4096
L_ = 50
NC_ = 2
NS_ = 16
NW_ = NC_ * NS_
BPW_ = B_ // NW_
LANES_ = 16
SEG_ = 128
VP_ = (VOCAB_ + 7) // 8 * 8
NSEG_ = 3 * VP_
NLCH_ = (L_ + LANES_ - 1) // LANES_
TAIL_START_ = D_ - LANES_
TAIL_SKIP_ = 288 - TAIL_START_


def _sc_body(table_hbm, tokens_hbm, out_hbm, idx_v, idx_a, idx_b, idx_c,
             rows_v, tbuf, sem):
    c = lax.axis_index("c")
    s = lax.axis_index("s")
    wid = s * NC_ + c
    base_b = wid * BPW_

    pltpu.sync_copy(tokens_hbm.at[pl.ds(base_b, BPW_)], idx_v)

    iota = lax.iota(jnp.int32, LANES_)
    iota_x_l = iota * L_
    tail_mask = iota >= TAIL_SKIP_

    def batch_body(i, _):
        for k in (0, LANES_, 2 * LANES_, L_ - LANES_):
            t = idx_v[i, pl.ds(k, LANES_)]
            s3 = t * 3
            sl = pl.ds(k, LANES_)
            idx_a[sl] = s3
            idx_b[sl] = s3 + 1
            idx_c[sl] = s3 + 2

        cp_a = pltpu.async_copy(table_hbm.at[idx_a], rows_v.at[pl.ds(0, L_)], sem)
        cp_b = pltpu.async_copy(table_hbm.at[idx_b], rows_v.at[pl.ds(L_, L_)], sem)
        cp_c = pltpu.async_copy(table_hbm.at[idx_c], rows_v.at[pl.ds(2 * L_, L_)], sem)
        cp_a.wait()
        cp_b.wait()
        cp_c.wait()

        def l_body(l, _):
            for j in range(18):
                d0 = j * LANES_
                blk, off = divmod(d0, SEG_)
                v = rows_v[blk * L_ + l, pl.ds(off, LANES_)]
                oidx = iota_x_l + (d0 * L_ + l)
                plsc.store_scatter(tbuf, [oidx], v)
            v = rows_v[2 * L_ + l, pl.ds(TAIL_START_ - 2 * SEG_, LANES_)]
            oidx = iota_x_l + (TAIL_START_ * L_ + l)
            plsc.store_scatter(tbuf, [oidx], v, mask=tail_mask)
            return 0

        lax.fori_loop(0, L_, l_body, 0)

        pltpu.sync_copy(tbuf, out_hbm.at[pl.ds((base_b + i) * D_ * L_, D_ * L_)])
        return 0

    lax.fori_loop(0, BPW_, batch_body, 0)


def kernel(tokens, word_embd_weight):
    tokens = tokens.reshape(B_, L_).astype(jnp.int32)
    tab = jnp.pad(word_embd_weight, ((0, VP_ - VOCAB_), (0, 384 - D_)))
    tab = tab.reshape(NSEG_, SEG_)
    mesh = plsc.VectorSubcoreMesh(core_axis_name="c", subcore_axis_name="s")
    out = pl.kernel(
        _sc_body,
        out_type=jax.ShapeDtypeStruct((B_ * D_ * L_,), jnp.float32),
        mesh=mesh,
        scratch_types=[
            pltpu.VMEM((BPW_, L_), jnp.int32),
            pltpu.VMEM((L_,), jnp.int32),
            pltpu.VMEM((L_,), jnp.int32),
            pltpu.VMEM((L_,), jnp.int32),
            pltpu.VMEM((3 * L_, SEG_), jnp.float32),
            pltpu.VMEM((D_ * L_,), jnp.float32),
            pltpu.SemaphoreType.DMA,
        ],
        compiler_params=pltpu.CompilerParams(
            needs_layout_passes=False, use_tc_tiling_on_sc=False
        ),
        name="embed_gather_transpose",
    )(tab, tokens)
    return out.reshape(B_, D_, L_, 1)
